# baseline (speedup 1.0000x reference)
import functools

import jax
import jax.numpy as jnp
from jax import lax
from jax.experimental import pallas as pl
from jax.experimental.pallas import tpu as pltpu
from jax.experimental.pallas import tpu_sc as plsc

EPS = 1e-5
MAX_NORM = 1.0 - 1e-5


def _norm(x):
    return jnp.sqrt(jnp.sum(x * x, axis=-1, keepdims=True))


def _atanh(n):
    return 0.5 * jnp.log((1.0 + n) / (1.0 - n))


def _log0(x):
    n = jnp.clip(_norm(x), EPS, MAX_NORM)
    return _atanh(n) * x / n


def _exp0(v):
    n = jnp.maximum(_norm(v), EPS)
    return jnp.tanh(n) * v / n


def _proj(x):
    n = jnp.maximum(_norm(x), EPS)
    return jnp.where(n > MAX_NORM, x / n * MAX_NORM, x)


def _mobius_matmul_log(h, W):
    v = jnp.dot(_log0(h), W, preferred_element_type=jnp.float32)
    return _log0(_proj(_exp0(v)))


def _perm_for_sc(d):
    rows = lax.broadcasted_iota(jnp.int32, (d, d), 0)
    cols = lax.broadcasted_iota(jnp.int32, (d, d), 1)
    sigma = cols // 2 + (d // 2) * (cols % 2)
    return jnp.where(rows == sigma, 1.0, 0.0).astype(jnp.float32)


def _to_sc_table(u):
    d = u.shape[-1]
    t = jnp.dot(u, _perm_for_sc(d), preferred_element_type=jnp.float32)
    return t.astype(jnp.bfloat16)


_ROWS_BLK = 2048


def _tc_pre_body(x_ref, w_ref, o_ref):
    o_ref[...] = _to_sc_table(_mobius_matmul_log(x_ref[...], w_ref[...]))


def _tc_pre(x, W):
    n, d = x.shape
    blk = min(_ROWS_BLK, n)
    return pl.pallas_call(
        _tc_pre_body,
        grid=(pl.cdiv(n, blk),),
        in_specs=[
            pl.BlockSpec((blk, d), lambda i: (i, 0)),
            pl.BlockSpec((d, d), lambda i: (0, 0)),
        ],
        out_specs=pl.BlockSpec((blk, d), lambda i: (i, 0)),
        out_shape=jax.ShapeDtypeStruct((n, d), jnp.bfloat16),
    )(x, W)


def _tc_mid_body(ap_ref, dp_ref, w_ref, u2_ref, deg_ref):
    ap = ap_ref[...]
    a = ap[0] + ap[1]
    nw = dp_ref.shape[0]
    deg = lax.dot_general(dp_ref[...], jnp.ones((nw, 1), jnp.float32),
                          (((0,), (0,)), ((), ())),
                          preferred_element_type=jnp.float32)
    deg = jnp.maximum(deg, 1.0)
    h = _proj(_exp0(jnp.tanh(a / deg)))
    u2_ref[...] = _to_sc_table(_mobius_matmul_log(h, w_ref[...]))
    deg_ref[...] = deg


def _tc_mid(aggp, degp, W, n):
    d = aggp.shape[2]
    nw = degp.shape[0]
    blk = min(_ROWS_BLK, n)
    return pl.pallas_call(
        _tc_mid_body,
        grid=(pl.cdiv(n, blk),),
        in_specs=[
            pl.BlockSpec((2, blk, d), lambda i: (0, i, 0)),
            pl.BlockSpec((nw, blk), lambda i: (0, i)),
            pl.BlockSpec((d, d), lambda i: (0, 0)),
        ],
        out_specs=[
            pl.BlockSpec((blk, d), lambda i: (i, 0)),
            pl.BlockSpec((blk, 1), lambda i: (i, 0)),
        ],
        out_shape=[
            jax.ShapeDtypeStruct((n, d), jnp.bfloat16),
            jax.ShapeDtypeStruct((n, 1), jnp.float32),
        ],
    )(aggp, degp, W)


def _tc_post_body(bp_ref, deg_ref, x_ref, o_ref):
    bp = bp_ref[...]
    a = bp[0] + bp[1]
    h = _proj(_exp0(jnp.tanh(a / deg_ref[...])))
    x = x_ref[...]
    h2 = jnp.sum(h * h, axis=-1, keepdims=True)
    x2 = jnp.sum(x * x, axis=-1, keepdims=True)
    hx = jnp.sum(h * x, axis=-1, keepdims=True)
    num = (1.0 + 2.0 * hx + x2) * h + (1.0 - h2) * x
    den = 1.0 + 2.0 * hx + h2 * x2
    o_ref[...] = _proj(num / jnp.maximum(den, EPS))


def _tc_post(aggp2, deg, x):
    n, d = x.shape
    blk = min(_ROWS_BLK, n)
    return pl.pallas_call(
        _tc_post_body,
        grid=(pl.cdiv(n, blk),),
        in_specs=[
            pl.BlockSpec((2, blk, d), lambda i: (0, i, 0)),
            pl.BlockSpec((blk, 1), lambda i: (i, 0)),
            pl.BlockSpec((blk, d), lambda i: (i, 0)),
        ],
        out_specs=pl.BlockSpec((blk, d), lambda i: (i, 0)),
        out_shape=jax.ShapeDtypeStruct((n, d), jnp.float32),
    )(aggp2, deg, x)


_NC = 2
_NS = 16
_CHUNK = 128


@functools.lru_cache(maxsize=None)
def _make_sc_agg(n, e_pad, d, with_deg):
    nw = _NC * _NS
    epw = e_pad // nw
    iters = epw // _CHUNK
    assert epw * nw == e_pad and iters * _CHUNK == epw
    assert iters % 2 == 1 and iters >= 3
    i0 = int(2 * iters * 0.76) | 1
    i1 = 2 * iters - i0
    assert i1 >= 3 and i0 % 2 == 1 and i1 % 2 == 1
    n_pad = ((n + 127) // 128) * 128
    rpt = n_pad // _NS

    mesh = plsc.VectorSubcoreMesh(core_axis_name="c", subcore_axis_name="s")
    out_type = [jax.ShapeDtypeStruct((_NC, n_pad, d), jnp.float32)]
    if with_deg:
        out_type.append(jax.ShapeDtypeStruct((nw, n_pad), jnp.float32))

    @functools.partial(
        pl.kernel,
        out_type=out_type,
        mesh=mesh,
        scratch_types=[
            pltpu.VMEM((2, 4, _CHUNK // 2), jnp.int32),
            pltpu.VMEM((_CHUNK, d), jnp.bfloat16),
            pltpu.VMEM((_CHUNK, d), jnp.bfloat16),
            pltpu.VMEM((_CHUNK, d), jnp.float32),
            pltpu.VMEM((n_pad,), jnp.float32),
            pltpu.VMEM_SHARED((n_pad, d), jnp.float32),
            pltpu.SemaphoreType.DMA,
            pltpu.SemaphoreType.DMA,
            pltpu.SemaphoreType.DMA,
            pltpu.SemaphoreType.DMA,
            pltpu.SemaphoreType.DMA,
            pltpu.SemaphoreType.DMA,
        ],
        compiler_params=pltpu.CompilerParams(use_tc_tiling_on_sc=False,
                                            needs_layout_passes=False),
    )
    def sc_agg(u_hbm, idx_hbm, zero_hbm, *rest):
        if with_deg:
            out_hbm, deg_hbm, ibuf, rows0, rows1, frows, degloc, agg_sh, \
                si0, si1, sr0, sr1, sr0b, sr1b = rest
        else:
            out_hbm, ibuf, rows0, rows1, frows, degloc, agg_sh, \
                si0, si1, sr0, sr1, sr0b, sr1b = rest
        c = lax.axis_index("c")
        s = lax.axis_index("s")
        r0 = s * rpt
        wid = s * _NC + c
        iters_c = jnp.where(c == 0, i0, i1)
        base = jnp.where(c == 0, s * i0, _NS * i0 + s * i1)
        last = nw * iters - 1

        def row(j):
            return jnp.minimum(base + j, last)

        pltpu.sync_copy(zero_hbm.at[pl.ds(r0, rpt)], agg_sh.at[pl.ds(r0, rpt)])
        if with_deg:
            def zbody(i, carry):
                degloc[pl.ds(i * 16, 16)] = jnp.zeros((16,), jnp.float32)
                return carry
            lax.fori_loop(0, n_pad // 16, zbody, 0)
        plsc.subcore_barrier()

        ones16 = jnp.ones((16,), jnp.float32)
        sems_i = (si0, si1)
        sems_r = ((sr0, sr0b), (sr1, sr1b))
        rows = (rows0, rows1)
        half = _CHUNK // 2

        def idx_wait(slot):
            pltpu.make_async_copy(idx_hbm.at[row(0)], ibuf.at[slot],
                                  sems_i[slot]).wait()

        def gather_start(j, slot):
            pltpu.async_copy(u_hbm.at[ibuf.at[slot, 0]],
                             rows[slot].at[pl.ds(0, half)], sems_r[slot][0])
            pltpu.async_copy(u_hbm.at[ibuf.at[slot, 1]],
                             rows[slot].at[pl.ds(half, half)],
                             sems_r[slot][1])

        dq = d // 32
        mask_hi = jnp.full((16,), -65536, jnp.int32)

        def consume(j, slot):
            pltpu.make_async_copy(u_hbm.at[ibuf.at[slot, 0]],
                                 rows[slot].at[pl.ds(0, half)],
                                 sems_r[slot][0]).wait()
            pltpu.make_async_copy(u_hbm.at[ibuf.at[slot, 1]],
                                 rows[slot].at[pl.ds(half, half)],
                                 sems_r[slot][1]).wait()
            rb = rows[slot]

            def convert_row(r, carry):
                for q in range(dq):
                    w = plsc.bitcast(rb[r, pl.ds(32 * q, 32)], jnp.int32)
                    frows[r, pl.ds(16 * q, 16)] = plsc.bitcast(
                        lax.shift_left(w, 16), jnp.float32)
                    frows[r, pl.ds(d // 2 + 16 * q, 16)] = plsc.bitcast(
                        w & mask_hi, jnp.float32)
                return carry

            lax.fori_loop(0, _CHUNK, convert_row, 0)
            pltpu.sync_copy(frows.at[pl.ds(0, half)],
                            agg_sh.at[ibuf.at[slot, 2]], add=True)
            pltpu.sync_copy(frows.at[pl.ds(half, half)],
                            agg_sh.at[ibuf.at[slot, 3]], add=True)
            if with_deg:
                for r2 in (2, 3):
                    for q in range(half // 16):
                        idx16 = ibuf[slot, r2, pl.ds(q * 16, 16)]
                        plsc.addupdate_scatter(degloc, [idx16], ones16)

        def phase(j, slot):
            other = 1 - slot
            idx_wait(other)
            gather_start(j + 1, other)
            consume(j, slot)
            pltpu.async_copy(idx_hbm.at[row(j + 2)], ibuf.at[slot],
                             sems_i[slot])

        pltpu.sync_copy(idx_hbm.at[row(0)], ibuf.at[0])
        pltpu.async_copy(idx_hbm.at[row(1)], ibuf.at[1], si1)
        gather_start(0, 0)

        def body(i2, carry):
            j = 2 * i2
            phase(j, 0)
            phase(j + 1, 1)
            return carry

        lax.fori_loop(0, (iters_c - 1) // 2, body, 0)
        idx_wait(1)
        consume(iters_c - 1, 0)

        if with_deg:
            pltpu.sync_copy(degloc, deg_hbm.at[wid])
        plsc.subcore_barrier()
        pltpu.sync_copy(agg_sh.at[pl.ds(r0, rpt)],
                        out_hbm.at[c, pl.ds(r0, rpt)])

    return sc_agg


def kernel(x, edge_index, W):
    n, d = x.shape
    e = edge_index.shape[1]
    src = edge_index[0].astype(jnp.int32)
    dst = edge_index[1].astype(jnp.int32)

    n_pad = ((n + 127) // 128) * 128
    nw = _NC * _NS
    iters = -(-e // (nw * _CHUNK))
    if iters % 2 == 0:
        iters += 1
    e_pad = nw * _CHUNK * iters
    pad = e_pad - e
    src_p = jnp.concatenate([src, jnp.zeros((pad,), jnp.int32)])
    dst_p = jnp.concatenate([dst, jnp.full((pad,), n_pad - 1, jnp.int32)])
    idx2 = jnp.stack([src_p.reshape(nw * iters, _CHUNK),
                      dst_p.reshape(nw * iters, _CHUNK)],
                     axis=1).reshape(nw * iters, 4, _CHUNK // 2)

    zeros = jnp.zeros((n_pad, d), jnp.float32)
    u1 = _tc_pre(x, W)
    aggp1, degp = _make_sc_agg(n, e_pad, d, True)(u1, idx2, zeros)
    u2, deg = _tc_mid(aggp1, degp, W, n)
    aggp2, = _make_sc_agg(n, e_pad, d, False)(u2, idx2, zeros)
    return _tc_post(aggp2, deg, x)


# speedup vs baseline: 2.1102x; 2.1102x over previous
import functools

import jax
import jax.numpy as jnp
from jax import lax
from jax.experimental import pallas as pl
from jax.experimental.pallas import tpu as pltpu
from jax.experimental.pallas import tpu_sc as plsc

EPS = 1e-5
MAX_NORM = 1.0 - 1e-5


def _norm(x):
    return jnp.sqrt(jnp.sum(x * x, axis=-1, keepdims=True))


def _atanh(n):
    return 0.5 * jnp.log((1.0 + n) / (1.0 - n))


def _log0(x):
    n = jnp.clip(_norm(x), EPS, MAX_NORM)
    return _atanh(n) * x / n


def _exp0(v):
    n = jnp.maximum(_norm(v), EPS)
    return jnp.tanh(n) * v / n


def _proj(x):
    n = jnp.maximum(_norm(x), EPS)
    return jnp.where(n > MAX_NORM, x / n * MAX_NORM, x)


def _mobius_matmul_log(h, W):
    v = jnp.dot(_log0(h), W, preferred_element_type=jnp.float32)
    return _log0(_proj(_exp0(v)))


def _perm_for_sc(d):
    rows = lax.broadcasted_iota(jnp.int32, (d, d), 0)
    cols = lax.broadcasted_iota(jnp.int32, (d, d), 1)
    sigma = cols // 2 + (d // 2) * (cols % 2)
    return jnp.where(rows == sigma, 1.0, 0.0).astype(jnp.float32)


def _to_sc_table(u):
    d = u.shape[-1]
    t = jnp.dot(u, _perm_for_sc(d), preferred_element_type=jnp.float32)
    return t.astype(jnp.bfloat16)


_ROWS_BLK = 2048


def _tc_pre_body(x_ref, w_ref, o_ref):
    o_ref[...] = _to_sc_table(_mobius_matmul_log(x_ref[...], w_ref[...]))


def _tc_pre(x, W):
    n, d = x.shape
    blk = min(_ROWS_BLK, n)
    return pl.pallas_call(
        _tc_pre_body,
        grid=(pl.cdiv(n, blk),),
        in_specs=[
            pl.BlockSpec((blk, d), lambda i: (i, 0)),
            pl.BlockSpec((d, d), lambda i: (0, 0)),
        ],
        out_specs=pl.BlockSpec((blk, d), lambda i: (i, 0)),
        out_shape=jax.ShapeDtypeStruct((n, d), jnp.bfloat16),
    )(x, W)


def _tc_mid_body(ap_ref, dp_ref, w_ref, u2_ref, deg_ref):
    ap = ap_ref[...]
    a = ap[0] + ap[1]
    nw = dp_ref.shape[0]
    deg = lax.dot_general(dp_ref[...], jnp.ones((nw, 1), jnp.float32),
                          (((0,), (0,)), ((), ())),
                          preferred_element_type=jnp.float32)
    deg = jnp.maximum(deg, 1.0)
    h = _proj(_exp0(jnp.tanh(a / deg)))
    u2_ref[...] = _to_sc_table(_mobius_matmul_log(h, w_ref[...]))
    deg_ref[...] = deg


def _tc_mid(aggp, degp, W, n):
    d = aggp.shape[2]
    nw = degp.shape[0]
    blk = min(_ROWS_BLK, n)
    return pl.pallas_call(
        _tc_mid_body,
        grid=(pl.cdiv(n, blk),),
        in_specs=[
            pl.BlockSpec((2, blk, d), lambda i: (0, i, 0)),
            pl.BlockSpec((nw, blk), lambda i: (0, i)),
            pl.BlockSpec((d, d), lambda i: (0, 0)),
        ],
        out_specs=[
            pl.BlockSpec((blk, d), lambda i: (i, 0)),
            pl.BlockSpec((blk, 1), lambda i: (i, 0)),
        ],
        out_shape=[
            jax.ShapeDtypeStruct((n, d), jnp.bfloat16),
            jax.ShapeDtypeStruct((n, 1), jnp.float32),
        ],
    )(aggp, degp, W)


def _tc_post_body(bp_ref, deg_ref, x_ref, o_ref):
    bp = bp_ref[...]
    a = bp[0] + bp[1]
    h = _proj(_exp0(jnp.tanh(a / deg_ref[...])))
    x = x_ref[...]
    h2 = jnp.sum(h * h, axis=-1, keepdims=True)
    x2 = jnp.sum(x * x, axis=-1, keepdims=True)
    hx = jnp.sum(h * x, axis=-1, keepdims=True)
    num = (1.0 + 2.0 * hx + x2) * h + (1.0 - h2) * x
    den = 1.0 + 2.0 * hx + h2 * x2
    o_ref[...] = _proj(num / jnp.maximum(den, EPS))


def _tc_post(aggp2, deg, x):
    n, d = x.shape
    blk = min(_ROWS_BLK, n)
    return pl.pallas_call(
        _tc_post_body,
        grid=(pl.cdiv(n, blk),),
        in_specs=[
            pl.BlockSpec((2, blk, d), lambda i: (0, i, 0)),
            pl.BlockSpec((blk, 1), lambda i: (i, 0)),
            pl.BlockSpec((blk, d), lambda i: (i, 0)),
        ],
        out_specs=pl.BlockSpec((blk, d), lambda i: (i, 0)),
        out_shape=jax.ShapeDtypeStruct((n, d), jnp.float32),
    )(aggp2, deg, x)


_NC = 2
_NS = 16
_CHUNK = 96


@functools.lru_cache(maxsize=None)
def _make_sc_agg(n, e_pad, d, with_deg):
    nw = _NC * _NS
    epw = e_pad // nw
    iters = epw // _CHUNK
    assert epw * nw == e_pad and iters * _CHUNK == epw
    assert iters % 2 == 1 and iters >= 3
    i0 = int(2 * iters * 0.5) | 1
    i1 = 2 * iters - i0
    assert i1 >= 3 and i0 % 2 == 1 and i1 % 2 == 1
    n_pad = ((n + 127) // 128) * 128
    rpt = n_pad // _NS

    mesh = plsc.VectorSubcoreMesh(core_axis_name="c", subcore_axis_name="s")
    out_type = [jax.ShapeDtypeStruct((_NC, n_pad, d), jnp.float32)]
    if with_deg:
        out_type.append(jax.ShapeDtypeStruct((nw, n_pad), jnp.float32))

    @functools.partial(
        pl.kernel,
        out_type=out_type,
        mesh=mesh,
        scratch_types=[
            pltpu.VMEM((2, 4, _CHUNK // 2), jnp.int32),
            pltpu.VMEM((_CHUNK, d), jnp.bfloat16),
            pltpu.VMEM((_CHUNK, d), jnp.bfloat16),
            pltpu.VMEM((_CHUNK, d), jnp.float32),
            pltpu.VMEM((_CHUNK, d), jnp.float32),
            pltpu.VMEM((n_pad,), jnp.float32),
            pltpu.VMEM_SHARED((n_pad, d), jnp.float32),
            pltpu.SemaphoreType.DMA,
            pltpu.SemaphoreType.DMA,
            pltpu.SemaphoreType.DMA,
            pltpu.SemaphoreType.DMA,
            pltpu.SemaphoreType.DMA,
            pltpu.SemaphoreType.DMA,
            pltpu.SemaphoreType.DMA,
            pltpu.SemaphoreType.DMA,
        ],
        compiler_params=pltpu.CompilerParams(use_tc_tiling_on_sc=False,
                                            needs_layout_passes=False),
    )
    def sc_agg(u_hbm, idx_hbm, zero_hbm, *rest):
        if with_deg:
            out_hbm, deg_hbm, ibuf, rows0, rows1, frows0, frows1, degloc, \
                agg_sh, si0, si1, sr0, sr1, sr0b, sr1b, ss0, ss1 = rest
        else:
            out_hbm, ibuf, rows0, rows1, frows0, frows1, degloc, \
                agg_sh, si0, si1, sr0, sr1, sr0b, sr1b, ss0, ss1 = rest
        c = lax.axis_index("c")
        s = lax.axis_index("s")
        r0 = s * rpt
        wid = s * _NC + c
        iters_c = jnp.where(c == 0, i0, i1)
        base = jnp.where(c == 0, s * i0, _NS * i0 + s * i1)
        last = nw * iters - 1

        def row(j):
            return jnp.minimum(base + j, last)

        pltpu.sync_copy(zero_hbm.at[pl.ds(r0, rpt)], agg_sh.at[pl.ds(r0, rpt)])
        if with_deg:
            def zbody(i, carry):
                degloc[pl.ds(i * 16, 16)] = jnp.zeros((16,), jnp.float32)
                return carry
            lax.fori_loop(0, n_pad // 16, zbody, 0)
        plsc.subcore_barrier()

        ones16 = jnp.ones((16,), jnp.float32)
        sems_i = (si0, si1)
        sems_r = ((sr0, sr0b), (sr1, sr1b))
        rows = (rows0, rows1)
        frows = (frows0, frows1)
        half = _CHUNK // 2

        def idx_wait(slot):
            pltpu.make_async_copy(idx_hbm.at[row(0)], ibuf.at[slot],
                                  sems_i[slot]).wait()

        def gather_start(j, slot):
            pltpu.async_copy(u_hbm.at[ibuf.at[slot, 0]],
                             rows[slot].at[pl.ds(0, half)], sems_r[slot][0])
            pltpu.async_copy(u_hbm.at[ibuf.at[slot, 1]],
                             rows[slot].at[pl.ds(half, half)],
                             sems_r[slot][1])

        dq = d // 32
        mask_hi = jnp.full((16,), -65536, jnp.int32)

        def drain_scatters(slot):
            pltpu.make_async_copy(frows[slot].at[pl.ds(0, half)],
                                 agg_sh.at[ibuf.at[slot, 2]], ss0).wait()
            pltpu.make_async_copy(frows[slot].at[pl.ds(half, half)],
                                 agg_sh.at[ibuf.at[slot, 3]], ss1).wait()

        def consume(j, slot):
            pltpu.make_async_copy(u_hbm.at[ibuf.at[slot, 0]],
                                 rows[slot].at[pl.ds(0, half)],
                                 sems_r[slot][0]).wait()
            pltpu.make_async_copy(u_hbm.at[ibuf.at[slot, 1]],
                                 rows[slot].at[pl.ds(half, half)],
                                 sems_r[slot][1]).wait()

            @pl.when(j > 0)
            def _():
                drain_scatters(slot)
            rb = rows[slot]
            fr = frows[slot]

            @plsc.parallel_loop(0, _CHUNK, unroll=4)
            def convert_row(r):
                for q in range(dq):
                    w = plsc.bitcast(rb[r, pl.ds(32 * q, 32)], jnp.int32)
                    fr[r, pl.ds(16 * q, 16)] = plsc.bitcast(
                        lax.shift_left(w, 16), jnp.float32)
                    fr[r, pl.ds(d // 2 + 16 * q, 16)] = plsc.bitcast(
                        w & mask_hi, jnp.float32)

            pltpu.async_copy(fr.at[pl.ds(0, half)],
                             agg_sh.at[ibuf.at[slot, 2]], ss0, add=True)
            pltpu.async_copy(fr.at[pl.ds(half, half)],
                             agg_sh.at[ibuf.at[slot, 3]], ss1, add=True)
            if with_deg:
                for r2 in (2, 3):
                    for q in range(half // 16):
                        idx16 = ibuf[slot, r2, pl.ds(q * 16, 16)]
                        plsc.addupdate_scatter(degloc, [idx16], ones16)

        def phase(j, slot):
            other = 1 - slot
            idx_wait(other)
            gather_start(j + 1, other)
            consume(j, slot)
            pltpu.async_copy(idx_hbm.at[row(j + 2)], ibuf.at[slot],
                             sems_i[slot])

        pltpu.sync_copy(idx_hbm.at[row(0)], ibuf.at[0])
        pltpu.async_copy(idx_hbm.at[row(1)], ibuf.at[1], si1)
        gather_start(0, 0)

        def body(i2, carry):
            j = 2 * i2
            phase(j, 0)
            phase(j + 1, 1)
            return carry

        lax.fori_loop(0, (iters_c - 1) // 2, body, 0)
        idx_wait(1)
        consume(iters_c - 1, 0)
        drain_scatters(0)

        if with_deg:
            pltpu.sync_copy(degloc, deg_hbm.at[wid])
        plsc.subcore_barrier()
        pltpu.sync_copy(agg_sh.at[pl.ds(r0, rpt)],
                        out_hbm.at[c, pl.ds(r0, rpt)])

    return sc_agg


def kernel(x, edge_index, W):
    n, d = x.shape
    e = edge_index.shape[1]
    src = edge_index[0].astype(jnp.int32)
    dst = edge_index[1].astype(jnp.int32)

    n_pad = ((n + 127) // 128) * 128
    nw = _NC * _NS
    iters = -(-e // (nw * _CHUNK))
    if iters % 2 == 0:
        iters += 1
    e_pad = nw * _CHUNK * iters
    pad = e_pad - e
    src_p = jnp.concatenate([src, jnp.zeros((pad,), jnp.int32)])
    dst_p = jnp.concatenate([dst, jnp.full((pad,), n_pad - 1, jnp.int32)])
    idx2 = jnp.stack([src_p.reshape(nw * iters, _CHUNK),
                      dst_p.reshape(nw * iters, _CHUNK)],
                     axis=1).reshape(nw * iters, 4, _CHUNK // 2)

    zeros = jnp.zeros((n_pad, d), jnp.float32)
    u1 = _tc_pre(x, W)
    aggp1, degp = _make_sc_agg(n, e_pad, d, True)(u1, idx2, zeros)
    u2, deg = _tc_mid(aggp1, degp, W, n)
    aggp2, = _make_sc_agg(n, e_pad, d, False)(u2, idx2, zeros)
    return _tc_post(aggp2, deg, x)


# speedup vs baseline: 2.2705x; 1.0760x over previous
import functools

import jax
import jax.numpy as jnp
from jax import lax
from jax.experimental import pallas as pl
from jax.experimental.pallas import tpu as pltpu
from jax.experimental.pallas import tpu_sc as plsc

EPS = 1e-5
MAX_NORM = 1.0 - 1e-5


def _norm(x):
    return jnp.sqrt(jnp.sum(x * x, axis=-1, keepdims=True))


def _atanh(n):
    return 0.5 * jnp.log((1.0 + n) / (1.0 - n))


def _log0(x):
    n = jnp.clip(_norm(x), EPS, MAX_NORM)
    return _atanh(n) * x / n


def _exp0(v):
    n = jnp.maximum(_norm(v), EPS)
    return jnp.tanh(n) * v / n


def _proj(x):
    n = jnp.maximum(_norm(x), EPS)
    return jnp.where(n > MAX_NORM, x / n * MAX_NORM, x)


def _mobius_matmul_log(h, W):
    v = jnp.dot(_log0(h), W, preferred_element_type=jnp.float32)
    return _log0(_proj(_exp0(v)))


def _perm_for_sc(d):
    rows = lax.broadcasted_iota(jnp.int32, (d, d), 0)
    cols = lax.broadcasted_iota(jnp.int32, (d, d), 1)
    sigma = cols // 2 + (d // 2) * (cols % 2)
    return jnp.where(rows == sigma, 1.0, 0.0).astype(jnp.float32)


def _to_sc_table(u):
    d = u.shape[-1]
    t = jnp.dot(u, _perm_for_sc(d), preferred_element_type=jnp.float32)
    return t.astype(jnp.bfloat16)


_ROWS_BLK = 2048


def _tc_pre_body(x_ref, w_ref, o_ref):
    o_ref[...] = _to_sc_table(_mobius_matmul_log(x_ref[...], w_ref[...]))


def _tc_pre(x, W):
    n, d = x.shape
    blk = min(_ROWS_BLK, n)
    return pl.pallas_call(
        _tc_pre_body,
        grid=(pl.cdiv(n, blk),),
        in_specs=[
            pl.BlockSpec((blk, d), lambda i: (i, 0)),
            pl.BlockSpec((d, d), lambda i: (0, 0)),
        ],
        out_specs=pl.BlockSpec((blk, d), lambda i: (i, 0)),
        out_shape=jax.ShapeDtypeStruct((n, d), jnp.bfloat16),
    )(x, W)


def _tc_mid_body(ap_ref, dp_ref, w_ref, u2_ref, deg_ref):
    ap = ap_ref[...]
    a = ap[0] + ap[1]
    nw = dp_ref.shape[0]
    deg = lax.dot_general(dp_ref[...], jnp.ones((nw, 1), jnp.float32),
                          (((0,), (0,)), ((), ())),
                          preferred_element_type=jnp.float32)
    deg = jnp.maximum(deg, 1.0)
    h = _proj(_exp0(jnp.tanh(a / deg)))
    u2_ref[...] = _to_sc_table(_mobius_matmul_log(h, w_ref[...]))
    deg_ref[...] = deg


def _tc_mid(aggp, degp, W, n):
    d = aggp.shape[2]
    nw = degp.shape[0]
    blk = min(_ROWS_BLK, n)
    return pl.pallas_call(
        _tc_mid_body,
        grid=(pl.cdiv(n, blk),),
        in_specs=[
            pl.BlockSpec((2, blk, d), lambda i: (0, i, 0)),
            pl.BlockSpec((nw, blk), lambda i: (0, i)),
            pl.BlockSpec((d, d), lambda i: (0, 0)),
        ],
        out_specs=[
            pl.BlockSpec((blk, d), lambda i: (i, 0)),
            pl.BlockSpec((blk, 1), lambda i: (i, 0)),
        ],
        out_shape=[
            jax.ShapeDtypeStruct((n, d), jnp.bfloat16),
            jax.ShapeDtypeStruct((n, 1), jnp.float32),
        ],
    )(aggp, degp, W)


def _tc_post_body(bp_ref, deg_ref, x_ref, o_ref):
    bp = bp_ref[...]
    a = bp[0] + bp[1]
    h = _proj(_exp0(jnp.tanh(a / deg_ref[...])))
    x = x_ref[...]
    h2 = jnp.sum(h * h, axis=-1, keepdims=True)
    x2 = jnp.sum(x * x, axis=-1, keepdims=True)
    hx = jnp.sum(h * x, axis=-1, keepdims=True)
    num = (1.0 + 2.0 * hx + x2) * h + (1.0 - h2) * x
    den = 1.0 + 2.0 * hx + h2 * x2
    o_ref[...] = _proj(num / jnp.maximum(den, EPS))


def _tc_post(aggp2, deg, x):
    n, d = x.shape
    blk = min(_ROWS_BLK, n)
    return pl.pallas_call(
        _tc_post_body,
        grid=(pl.cdiv(n, blk),),
        in_specs=[
            pl.BlockSpec((2, blk, d), lambda i: (0, i, 0)),
            pl.BlockSpec((blk, 1), lambda i: (i, 0)),
            pl.BlockSpec((blk, d), lambda i: (i, 0)),
        ],
        out_specs=pl.BlockSpec((blk, d), lambda i: (i, 0)),
        out_shape=jax.ShapeDtypeStruct((n, d), jnp.float32),
    )(aggp2, deg, x)


_NC = 2
_NS = 16
_CHUNK = 96


@functools.lru_cache(maxsize=None)
def _make_sc_agg(n, e_pad, d, with_deg):
    nw = _NC * _NS
    epw = e_pad // nw
    iters = epw // _CHUNK
    assert epw * nw == e_pad and iters * _CHUNK == epw
    assert iters % 2 == 1 and iters >= 3
    i0 = int(2 * iters * 0.583) | 1
    i1 = 2 * iters - i0
    assert i1 >= 3 and i0 % 2 == 1 and i1 % 2 == 1
    n_pad = ((n + 127) // 128) * 128
    rpt = n_pad // _NS

    mesh = plsc.VectorSubcoreMesh(core_axis_name="c", subcore_axis_name="s")
    out_type = [jax.ShapeDtypeStruct((_NC, n_pad, d), jnp.float32)]
    if with_deg:
        out_type.append(jax.ShapeDtypeStruct((nw, n_pad), jnp.float32))

    @functools.partial(
        pl.kernel,
        out_type=out_type,
        mesh=mesh,
        scratch_types=[
            pltpu.VMEM((2, 4, _CHUNK // 2), jnp.int32),
            pltpu.VMEM((_CHUNK, d), jnp.bfloat16),
            pltpu.VMEM((_CHUNK, d), jnp.bfloat16),
            pltpu.VMEM((_CHUNK, d), jnp.float32),
            pltpu.VMEM((_CHUNK, d), jnp.float32),
            pltpu.VMEM((n_pad,), jnp.float32),
            pltpu.VMEM_SHARED((n_pad, d), jnp.float32),
            pltpu.SemaphoreType.DMA,
            pltpu.SemaphoreType.DMA,
            pltpu.SemaphoreType.DMA,
            pltpu.SemaphoreType.DMA,
            pltpu.SemaphoreType.DMA,
            pltpu.SemaphoreType.DMA,
            pltpu.SemaphoreType.DMA,
            pltpu.SemaphoreType.DMA,
        ],
        compiler_params=pltpu.CompilerParams(use_tc_tiling_on_sc=False,
                                            needs_layout_passes=False),
    )
    def sc_agg(u_hbm, idx_hbm, zero_hbm, *rest):
        if with_deg:
            out_hbm, deg_hbm, ibuf, rows0, rows1, frows0, frows1, degloc, \
                agg_sh, si0, si1, sr0, sr1, sr0b, sr1b, ss0, ss1 = rest
        else:
            out_hbm, ibuf, rows0, rows1, frows0, frows1, degloc, \
                agg_sh, si0, si1, sr0, sr1, sr0b, sr1b, ss0, ss1 = rest
        c = lax.axis_index("c")
        s = lax.axis_index("s")
        r0 = s * rpt
        wid = s * _NC + c
        iters_c = jnp.where(c == 0, i0, i1)
        base = jnp.where(c == 0, s * i0, _NS * i0 + s * i1)
        last = nw * iters - 1

        def row(j):
            return jnp.minimum(base + j, last)

        pltpu.sync_copy(zero_hbm.at[pl.ds(r0, rpt)], agg_sh.at[pl.ds(r0, rpt)])
        if with_deg:
            def zbody(i, carry):
                degloc[pl.ds(i * 16, 16)] = jnp.zeros((16,), jnp.float32)
                return carry
            lax.fori_loop(0, n_pad // 16, zbody, 0)
        plsc.subcore_barrier()

        ones16 = jnp.ones((16,), jnp.float32)
        sems_i = (si0, si1)
        sems_r = ((sr0, sr0b), (sr1, sr1b))
        rows = (rows0, rows1)
        frows = (frows0, frows1)
        half = _CHUNK // 2

        def idx_wait(slot):
            pltpu.make_async_copy(idx_hbm.at[row(0)], ibuf.at[slot],
                                  sems_i[slot]).wait()

        def gather_start(j, slot):
            pltpu.async_copy(u_hbm.at[ibuf.at[slot, 0]],
                             rows[slot].at[pl.ds(0, half)], sems_r[slot][0])
            pltpu.async_copy(u_hbm.at[ibuf.at[slot, 1]],
                             rows[slot].at[pl.ds(half, half)],
                             sems_r[slot][1])

        dq = d // 32
        mask_hi = jnp.full((16,), -65536, jnp.int32)

        def drain_scatters(slot):
            pltpu.make_async_copy(frows[slot].at[pl.ds(0, half)],
                                 agg_sh.at[ibuf.at[slot, 2]], ss0).wait()
            pltpu.make_async_copy(frows[slot].at[pl.ds(half, half)],
                                 agg_sh.at[ibuf.at[slot, 3]], ss1).wait()

        def consume(j, slot):
            pltpu.make_async_copy(u_hbm.at[ibuf.at[slot, 0]],
                                 rows[slot].at[pl.ds(0, half)],
                                 sems_r[slot][0]).wait()
            pltpu.make_async_copy(u_hbm.at[ibuf.at[slot, 1]],
                                 rows[slot].at[pl.ds(half, half)],
                                 sems_r[slot][1]).wait()

            @pl.when(j > 0)
            def _():
                drain_scatters(slot)
            rb = rows[slot]
            fr = frows[slot]

            @plsc.parallel_loop(0, _CHUNK, unroll=4)
            def convert_row(r):
                for q in range(dq):
                    w = plsc.bitcast(rb[r, pl.ds(32 * q, 32)], jnp.int32)
                    fr[r, pl.ds(16 * q, 16)] = plsc.bitcast(
                        lax.shift_left(w, 16), jnp.float32)
                    fr[r, pl.ds(d // 2 + 16 * q, 16)] = plsc.bitcast(
                        w & mask_hi, jnp.float32)

            pltpu.async_copy(fr.at[pl.ds(0, half)],
                             agg_sh.at[ibuf.at[slot, 2]], ss0, add=True)
            pltpu.async_copy(fr.at[pl.ds(half, half)],
                             agg_sh.at[ibuf.at[slot, 3]], ss1, add=True)
            if with_deg:
                for r2 in (2, 3):
                    for q in range(half // 16):
                        idx16 = ibuf[slot, r2, pl.ds(q * 16, 16)]
                        plsc.addupdate_scatter(degloc, [idx16], ones16)

        def phase(j, slot):
            other = 1 - slot
            idx_wait(other)
            gather_start(j + 1, other)
            consume(j, slot)
            pltpu.async_copy(idx_hbm.at[row(j + 2)], ibuf.at[slot],
                             sems_i[slot])

        pltpu.sync_copy(idx_hbm.at[row(0)], ibuf.at[0])
        pltpu.async_copy(idx_hbm.at[row(1)], ibuf.at[1], si1)
        gather_start(0, 0)

        def body(i2, carry):
            j = 2 * i2
            phase(j, 0)
            phase(j + 1, 1)
            return carry

        lax.fori_loop(0, (iters_c - 1) // 2, body, 0)
        idx_wait(1)
        consume(iters_c - 1, 0)
        drain_scatters(0)

        if with_deg:
            pltpu.sync_copy(degloc, deg_hbm.at[wid])
        plsc.subcore_barrier()
        pltpu.sync_copy(agg_sh.at[pl.ds(r0, rpt)],
                        out_hbm.at[c, pl.ds(r0, rpt)])

    return sc_agg


def kernel(x, edge_index, W):
    n, d = x.shape
    e = edge_index.shape[1]
    src = edge_index[0].astype(jnp.int32)
    dst = edge_index[1].astype(jnp.int32)

    n_pad = ((n + 127) // 128) * 128
    nw = _NC * _NS
    iters = -(-e // (nw * _CHUNK))
    if iters % 2 == 0:
        iters += 1
    e_pad = nw * _CHUNK * iters
    pad = e_pad - e
    src_p = jnp.concatenate([src, jnp.zeros((pad,), jnp.int32)])
    dst_p = jnp.concatenate([dst, jnp.full((pad,), n_pad - 1, jnp.int32)])
    idx2 = jnp.stack([src_p.reshape(nw * iters, _CHUNK),
                      dst_p.reshape(nw * iters, _CHUNK)],
                     axis=1).reshape(nw * iters, 4, _CHUNK // 2)

    zeros = jnp.zeros((n_pad, d), jnp.float32)
    u1 = _tc_pre(x, W)
    aggp1, degp = _make_sc_agg(n, e_pad, d, True)(u1, idx2, zeros)
    u2, deg = _tc_mid(aggp1, degp, W, n)
    aggp2, = _make_sc_agg(n, e_pad, d, False)(u2, idx2, zeros)
    return _tc_post(aggp2, deg, x)
